# baseline (speedup 1.0000x reference)
import functools

import jax
import jax.numpy as jnp
from jax import lax
from jax.experimental import pallas as pl
from jax.experimental.pallas import tpu as pltpu
from jax.experimental.pallas import tpu_sc as plsc

_NUM_CORES = 2
_NUM_SUBCORES = 16
_LANES = 16
_NBUF = 3


@functools.cache
def _make_sc_kernel(N: int, D: int, C: int):
    NW = _NUM_CORES * _NUM_SUBCORES
    NCHUNKS = N // C
    TMAX = (NCHUNKS + NW - 1) // NW
    NTB = (TMAX + 1 + _NBUF - 1) // _NBUF
    JV = D // _LANES

    mesh = plsc.VectorSubcoreMesh(
        core_axis_name="c", subcore_axis_name="s",
        num_cores=_NUM_CORES, num_subcores=_NUM_SUBCORES)

    def body(lb_hbm, ub_hbm, nid_hbm, out_hbm, idx_v, lb_v, ub_v,
             gsem, ssem, isem):
        wid = lax.axis_index("s") * _NUM_CORES + lax.axis_index("c")

        def in_range(t):
            return (t * NW + wid) < NCHUNKS

        def idx_desc(t, p):
            g = t * NW + wid
            return pltpu.make_async_copy(nid_hbm.at[pl.ds(g * C, C)],
                                         idx_v.at[p], isem.at[p])

        def start_idx(t, p):
            @pl.when(in_range(t))
            def _():
                idx_desc(t, p).start()

        def wait_idx(t, p):
            @pl.when(in_range(t))
            def _():
                idx_desc(t, p).wait()

        def gather_descs(p):
            return (pltpu.make_async_copy(lb_hbm.at[idx_v.at[p]], lb_v.at[p],
                                          gsem.at[p]),
                    pltpu.make_async_copy(ub_hbm.at[idx_v.at[p]], ub_v.at[p],
                                          gsem.at[p]))

        def scatter_descs(g, p):
            base = g * C
            return (pltpu.make_async_copy(lb_v.at[p],
                                          out_hbm.at[0, pl.ds(base, C)],
                                          ssem.at[p]),
                    pltpu.make_async_copy(ub_v.at[p],
                                          out_hbm.at[1, pl.ds(base, C)],
                                          ssem.at[p]))

        def start_gather(t, p):
            @pl.when(in_range(t))
            def _():
                for d in gather_descs(p):
                    d.start()

        def wait_gather(t, p):
            @pl.when(in_range(t))
            def _():
                for d in gather_descs(p):
                    d.wait()

        H0 = (C // 2 + 7) // 8 * 8
        SPLITS = ((0, H0), (H0, C))

        def start_scatter_half(t, p, h):
            @pl.when(in_range(t))
            def _():
                g = t * NW + wid
                lo, hi = SPLITS[h]
                for src, plane in ((lb_v, 0), (ub_v, 1)):
                    pltpu.make_async_copy(
                        src.at[p, pl.ds(lo, hi - lo)],
                        out_hbm.at[plane, pl.ds(g * C + lo, hi - lo)],
                        ssem.at[p]).start()

        def wait_scatter(t, p):
            @pl.when((t >= 0) & in_range(t))
            def _():
                g = t * NW + wid
                for d in scatter_descs(g, p):
                    d.wait()

        def compute_half(t, p, h):
            @pl.when(in_range(t))
            def _():
                def row_body(r, carry):
                    for j in range(JV):
                        sl = pl.ds(j * _LANES, _LANES)
                        l = lb_v[p, r, sl]
                        u = ub_v[p, r, sl]
                        zero = jnp.zeros_like(l)
                        ub_v[p, r, sl] = jnp.where(l == zero, zero,
                                                   jnp.maximum(u, zero))
                        lb_v[p, r, sl] = jnp.where((l + u) > zero, l, zero)
                    return carry

                lax.fori_loop(SPLITS[h][0], SPLITS[h][1], row_body, 0)

        start_idx(0, 0)
        start_idx(1, 1)
        start_idx(2, 2)
        wait_idx(0, 0)
        start_gather(0, 0)
        wait_idx(1, 1)
        start_gather(1, 1)

        def block(tb, carry):
            for p in range(_NBUF):
                t = tb * _NBUF + p
                wait_gather(t, p)
                pn = (p + 2) % _NBUF
                wait_scatter(t - 1, pn)
                wait_idx(t + 2, pn)
                start_gather(t + 2, pn)
                start_idx(t + 3, p)
                compute_half(t, p, 0)
                start_scatter_half(t, p, 0)
                compute_half(t, p, 1)
                start_scatter_half(t, p, 1)
            return carry

        lax.fori_loop(0, NTB, block, 0)

    return pl.kernel(
        body,
        out_type=jax.ShapeDtypeStruct((2, N, D), jnp.float32),
        mesh=mesh,
        scratch_types=[
            pltpu.VMEM((_NBUF, C), jnp.int32),
            pltpu.VMEM((_NBUF, C, D), jnp.float32),
            pltpu.VMEM((_NBUF, C, D), jnp.float32),
            pltpu.SemaphoreType.DMA((_NBUF,)),
            pltpu.SemaphoreType.DMA((_NBUF,)),
            pltpu.SemaphoreType.DMA((_NBUF,)),
        ],
    )


def kernel(lb, ub, node_id):
    N, D = lb.shape
    C = next(c for c in (40, 16, 8) if N % c == 0 and c * D * 4 * 2 * _NBUF <= 500_000)
    return _make_sc_kernel(N, D, C)(lb, ub, node_id.astype(jnp.int32))


# speedup vs baseline: 1.0439x; 1.0439x over previous
import functools

import jax
import jax.numpy as jnp
from jax import lax
from jax.experimental import pallas as pl
from jax.experimental.pallas import tpu as pltpu
from jax.experimental.pallas import tpu_sc as plsc

_NUM_CORES = 2
_NUM_SUBCORES = 16
_LANES = 16
_NBUF = 3


@functools.cache
def _make_sc_kernel(N: int, D: int, C: int):
    NW = _NUM_CORES * _NUM_SUBCORES
    NCHUNKS = N // C
    TMAX = (NCHUNKS + NW - 1) // NW
    NTB = (TMAX + 1 + _NBUF - 1) // _NBUF
    JV = D // _LANES

    mesh = plsc.VectorSubcoreMesh(
        core_axis_name="c", subcore_axis_name="s",
        num_cores=_NUM_CORES, num_subcores=_NUM_SUBCORES)

    def body(lb_hbm, ub_hbm, nid_hbm, out_hbm, idx_v, lb_v, ub_v,
             gsem, ssem, isem):
        wid = lax.axis_index("s") * _NUM_CORES + lax.axis_index("c")

        def in_range(t):
            return (t * NW + wid) < NCHUNKS

        def idx_desc(t, p):
            g = t * NW + wid
            return pltpu.make_async_copy(nid_hbm.at[pl.ds(g * C, C)],
                                         idx_v.at[p], isem.at[p])

        def start_idx(t, p):
            @pl.when(in_range(t))
            def _():
                idx_desc(t, p).start()

        def wait_idx(t, p):
            @pl.when(in_range(t))
            def _():
                idx_desc(t, p).wait()

        def gather_descs(p):
            return (pltpu.make_async_copy(lb_hbm.at[idx_v.at[p]], lb_v.at[p],
                                          gsem.at[p]),
                    pltpu.make_async_copy(ub_hbm.at[idx_v.at[p]], ub_v.at[p],
                                          gsem.at[p]))

        def scatter_descs(g, p):
            base = g * C
            return (pltpu.make_async_copy(lb_v.at[p],
                                          out_hbm.at[0, pl.ds(base, C)],
                                          ssem.at[p]),
                    pltpu.make_async_copy(ub_v.at[p],
                                          out_hbm.at[1, pl.ds(base, C)],
                                          ssem.at[p]))

        def start_gather(t, p):
            @pl.when(in_range(t))
            def _():
                for d in gather_descs(p):
                    d.start()

        def wait_gather(t, p):
            @pl.when(in_range(t))
            def _():
                for d in gather_descs(p):
                    d.wait()

        H0 = (C // 2 + 7) // 8 * 8
        SPLITS = ((0, H0), (H0, C))

        def start_scatter_half(t, p, h):
            @pl.when(in_range(t))
            def _():
                g = t * NW + wid
                lo, hi = SPLITS[h]
                for src, plane in ((lb_v, 0), (ub_v, 1)):
                    pltpu.make_async_copy(
                        src.at[p, pl.ds(lo, hi - lo)],
                        out_hbm.at[plane, pl.ds(g * C + lo, hi - lo)],
                        ssem.at[p]).start()

        def wait_scatter(t, p):
            @pl.when((t >= 0) & in_range(t))
            def _():
                g = t * NW + wid
                for d in scatter_descs(g, p):
                    d.wait()

        def compute_half(t, p, h):
            @pl.when(in_range(t))
            def _():
                def row_body(r, carry):
                    for j in range(JV):
                        sl = pl.ds(j * _LANES, _LANES)
                        l = lb_v[p, r, sl]
                        u = ub_v[p, r, sl]
                        zero = jnp.zeros_like(l)
                        ub_v[p, r, sl] = jnp.where(l == zero, zero,
                                                   jnp.maximum(u, zero))
                        lb_v[p, r, sl] = jnp.where((l + u) > zero, l, zero)
                    return carry

                lax.fori_loop(SPLITS[h][0], SPLITS[h][1], row_body, 0)

        start_idx(0, 0)
        start_idx(1, 1)
        start_idx(2, 2)
        wait_idx(0, 0)
        start_gather(0, 0)
        wait_idx(1, 1)
        start_gather(1, 1)

        def block(tb, carry):
            for p in range(_NBUF):
                t = tb * _NBUF + p
                wait_gather(t, p)
                compute_half(t, p, 0)
                start_scatter_half(t, p, 0)
                compute_half(t, p, 1)
                start_scatter_half(t, p, 1)
                pn = (p + 2) % _NBUF
                wait_scatter(t - 1, pn)
                wait_idx(t + 2, pn)
                start_gather(t + 2, pn)
                start_idx(t + 3, p)
            return carry

        lax.fori_loop(0, NTB, block, 0)

    return pl.kernel(
        body,
        out_type=jax.ShapeDtypeStruct((2, N, D), jnp.float32),
        mesh=mesh,
        scratch_types=[
            pltpu.VMEM((_NBUF, C), jnp.int32),
            pltpu.VMEM((_NBUF, C, D), jnp.float32),
            pltpu.VMEM((_NBUF, C, D), jnp.float32),
            pltpu.SemaphoreType.DMA((_NBUF,)),
            pltpu.SemaphoreType.DMA((_NBUF,)),
            pltpu.SemaphoreType.DMA((_NBUF,)),
        ],
    )


def kernel(lb, ub, node_id):
    N, D = lb.shape
    C = next(c for c in (40, 16, 8) if N % c == 0 and c * D * 4 * 2 * _NBUF <= 500_000)
    return _make_sc_kernel(N, D, C)(lb, ub, node_id.astype(jnp.int32))


# speedup vs baseline: 1.0810x; 1.0355x over previous
import functools

import jax
import jax.numpy as jnp
from jax import lax
from jax.experimental import pallas as pl
from jax.experimental.pallas import tpu as pltpu
from jax.experimental.pallas import tpu_sc as plsc

_NUM_CORES = 2
_NUM_SUBCORES = 16
_LANES = 16
_NBUF = 3


@functools.cache
def _make_sc_kernel(N: int, D: int, C: int):
    NW = _NUM_CORES * _NUM_SUBCORES
    NCHUNKS = N // C
    TMAX = (NCHUNKS + NW - 1) // NW
    NTB = (TMAX + 1 + _NBUF - 1) // _NBUF
    JV = D // _LANES

    mesh = plsc.VectorSubcoreMesh(
        core_axis_name="c", subcore_axis_name="s",
        num_cores=_NUM_CORES, num_subcores=_NUM_SUBCORES)

    def body(lb_hbm, ub_hbm, nid_hbm, out_hbm, idx_v, lb_v, ub_v,
             gsem, ssem, isem):
        wid = lax.axis_index("s") * _NUM_CORES + lax.axis_index("c")

        def in_range(t):
            return (t * NW + wid) < NCHUNKS

        def idx_desc(t, p):
            g = t * NW + wid
            return pltpu.make_async_copy(nid_hbm.at[pl.ds(g * C, C)],
                                         idx_v.at[p], isem.at[p])

        def start_idx(t, p):
            @pl.when(in_range(t))
            def _():
                idx_desc(t, p).start()

        def wait_idx(t, p):
            @pl.when(in_range(t))
            def _():
                idx_desc(t, p).wait()

        def gather_descs(p):
            return (pltpu.make_async_copy(lb_hbm.at[idx_v.at[p]], lb_v.at[p],
                                          gsem.at[p]),
                    pltpu.make_async_copy(ub_hbm.at[idx_v.at[p]], ub_v.at[p],
                                          gsem.at[p]))

        def scatter_descs(g, p):
            base = g * C
            return (pltpu.make_async_copy(lb_v.at[p],
                                          out_hbm.at[0, pl.ds(base, C)],
                                          ssem.at[p]),
                    pltpu.make_async_copy(ub_v.at[p],
                                          out_hbm.at[1, pl.ds(base, C)],
                                          ssem.at[p]))

        def start_gather(t, p):
            @pl.when(in_range(t))
            def _():
                for d in gather_descs(p):
                    d.start()

        def wait_gather(t, p):
            @pl.when(in_range(t))
            def _():
                for d in gather_descs(p):
                    d.wait()

        H0 = (C // 2 + 7) // 8 * 8
        SPLITS = tuple((lo, min(lo + 8, C)) for lo in range(0, C, 8)) \
            if C % 8 == 0 else ((0, H0), (H0, C))

        def start_scatter_half(t, p, h):
            @pl.when(in_range(t))
            def _():
                g = t * NW + wid
                lo, hi = SPLITS[h]
                for src, plane in ((lb_v, 0), (ub_v, 1)):
                    pltpu.make_async_copy(
                        src.at[p, pl.ds(lo, hi - lo)],
                        out_hbm.at[plane, pl.ds(g * C + lo, hi - lo)],
                        ssem.at[p]).start()

        def wait_scatter(t, p):
            @pl.when((t >= 0) & in_range(t))
            def _():
                g = t * NW + wid
                for d in scatter_descs(g, p):
                    d.wait()

        def compute_half(t, p, h):
            @pl.when(in_range(t))
            def _():
                def row_body(r, carry):
                    for j in range(JV):
                        sl = pl.ds(j * _LANES, _LANES)
                        l = lb_v[p, r, sl]
                        u = ub_v[p, r, sl]
                        zero = jnp.zeros_like(l)
                        ub_v[p, r, sl] = jnp.where(l == zero, zero,
                                                   jnp.maximum(u, zero))
                        lb_v[p, r, sl] = jnp.where((l + u) > zero, l, zero)
                    return carry

                lax.fori_loop(SPLITS[h][0], SPLITS[h][1], row_body, 0)

        start_idx(0, 0)
        start_idx(1, 1)
        start_idx(2, 2)
        wait_idx(0, 0)
        start_gather(0, 0)
        wait_idx(1, 1)
        start_gather(1, 1)

        def block(tb, carry):
            for p in range(_NBUF):
                t = tb * _NBUF + p
                wait_gather(t, p)
                for h in range(len(SPLITS)):
                    compute_half(t, p, h)
                    start_scatter_half(t, p, h)
                pn = (p + 2) % _NBUF
                wait_scatter(t - 1, pn)
                wait_idx(t + 2, pn)
                start_gather(t + 2, pn)
                start_idx(t + 3, p)
            return carry

        lax.fori_loop(0, NTB, block, 0)

    return pl.kernel(
        body,
        out_type=jax.ShapeDtypeStruct((2, N, D), jnp.float32),
        mesh=mesh,
        scratch_types=[
            pltpu.VMEM((_NBUF, C), jnp.int32),
            pltpu.VMEM((_NBUF, C, D), jnp.float32),
            pltpu.VMEM((_NBUF, C, D), jnp.float32),
            pltpu.SemaphoreType.DMA((_NBUF,)),
            pltpu.SemaphoreType.DMA((_NBUF,)),
            pltpu.SemaphoreType.DMA((_NBUF,)),
        ],
    )


def kernel(lb, ub, node_id):
    N, D = lb.shape
    C = next(c for c in (40, 16, 8) if N % c == 0 and c * D * 4 * 2 * _NBUF <= 500_000)
    return _make_sc_kernel(N, D, C)(lb, ub, node_id.astype(jnp.int32))


# speedup vs baseline: 1.1219x; 1.0379x over previous
import functools

import jax
import jax.numpy as jnp
from jax import lax
from jax.experimental import pallas as pl
from jax.experimental.pallas import tpu as pltpu
from jax.experimental.pallas import tpu_sc as plsc

_NUM_CORES = 2
_NUM_SUBCORES = 16
_LANES = 16
_NBUF = 3


@functools.cache
def _make_sc_kernel(N: int, D: int, C: int):
    NW = _NUM_CORES * _NUM_SUBCORES
    NCHUNKS = N // C
    TMAX = (NCHUNKS + NW - 1) // NW
    NTB = (TMAX + 1 + _NBUF - 1) // _NBUF
    JV = D // _LANES

    mesh = plsc.VectorSubcoreMesh(
        core_axis_name="c", subcore_axis_name="s",
        num_cores=_NUM_CORES, num_subcores=_NUM_SUBCORES)

    def body(lb_hbm, ub_hbm, nid_hbm, out_hbm, idx_v, lb_v, ub_v,
             gsem, ssem, isem):
        wid = lax.axis_index("s") * _NUM_CORES + lax.axis_index("c")

        def in_range(t):
            return (t * NW + wid) < NCHUNKS

        def idx_desc(t, p):
            g = t * NW + wid
            return pltpu.make_async_copy(nid_hbm.at[pl.ds(g * C, C)],
                                         idx_v.at[p], isem.at[p])

        def start_idx(t, p):
            @pl.when(in_range(t))
            def _():
                idx_desc(t, p).start()

        def wait_idx(t, p):
            @pl.when(in_range(t))
            def _():
                idx_desc(t, p).wait()

        def gather_descs(p):
            return (pltpu.make_async_copy(lb_hbm.at[idx_v.at[p]], lb_v.at[p],
                                          gsem.at[p]),
                    pltpu.make_async_copy(ub_hbm.at[idx_v.at[p]], ub_v.at[p],
                                          gsem.at[p]))

        def scatter_descs(g, p):
            base = g * C
            return (pltpu.make_async_copy(lb_v.at[p],
                                          out_hbm.at[0, pl.ds(base, C)],
                                          ssem.at[p]),
                    pltpu.make_async_copy(ub_v.at[p],
                                          out_hbm.at[1, pl.ds(base, C)],
                                          ssem.at[p]))

        def start_gather(t, p):
            @pl.when(in_range(t))
            def _():
                for d in gather_descs(p):
                    d.start()

        def wait_gather(t, p):
            @pl.when(in_range(t))
            def _():
                for d in gather_descs(p):
                    d.wait()

        H0 = (C // 2 + 7) // 8 * 8
        SPLITS = tuple((lo, min(lo + 8, C)) for lo in range(0, C, 8)) \
            if C % 8 == 0 else ((0, H0), (H0, C))

        def start_scatter_half(t, p, h):
            @pl.when(in_range(t))
            def _():
                g = t * NW + wid
                lo, hi = SPLITS[h]
                for src, plane in ((lb_v, 0), (ub_v, 1)):
                    pltpu.make_async_copy(
                        src.at[p, pl.ds(lo, hi - lo)],
                        out_hbm.at[plane, pl.ds(g * C + lo, hi - lo)],
                        ssem.at[p]).start()

        def wait_scatter(t, p):
            @pl.when((t >= 0) & in_range(t))
            def _():
                g = t * NW + wid
                for d in scatter_descs(g, p):
                    d.wait()

        def compute_half(t, p, h):
            @pl.when(in_range(t))
            def _():
                def row_body(r, carry):
                    for j in range(JV):
                        sl = pl.ds(j * _LANES, _LANES)
                        l = lb_v[p, r, sl]
                        u = ub_v[p, r, sl]
                        zero = jnp.zeros_like(l)
                        ub_v[p, r, sl] = jnp.where(l == zero, zero,
                                                   jnp.maximum(u, zero))
                        lb_v[p, r, sl] = jnp.where((l + u) > zero, l, zero)
                    return carry

                lax.fori_loop(SPLITS[h][0], SPLITS[h][1], row_body, 0)

        start_idx(0, 0)
        start_idx(1, 1)
        start_idx(2, 2)
        wait_idx(0, 0)
        start_gather(0, 0)
        wait_idx(1, 1)
        start_gather(1, 1)

        def block(tb, carry):
            for p in range(_NBUF):
                t = tb * _NBUF + p
                wait_gather(t, p)
                for h in range(len(SPLITS)):
                    compute_half(t, p, h)
                    start_scatter_half(t, p, h)
                    if h == 0:
                        pn = (p + 2) % _NBUF
                        wait_scatter(t - 1, pn)
                        wait_idx(t + 2, pn)
                        start_gather(t + 2, pn)
                        start_idx(t + 3, p)
            return carry

        lax.fori_loop(0, NTB, block, 0)

    return pl.kernel(
        body,
        out_type=jax.ShapeDtypeStruct((2, N, D), jnp.float32),
        mesh=mesh,
        scratch_types=[
            pltpu.VMEM((_NBUF, C), jnp.int32),
            pltpu.VMEM((_NBUF, C, D), jnp.float32),
            pltpu.VMEM((_NBUF, C, D), jnp.float32),
            pltpu.SemaphoreType.DMA((_NBUF,)),
            pltpu.SemaphoreType.DMA((_NBUF,)),
            pltpu.SemaphoreType.DMA((_NBUF,)),
        ],
    )


def kernel(lb, ub, node_id):
    N, D = lb.shape
    C = next(c for c in (40, 16, 8) if N % c == 0 and c * D * 4 * 2 * _NBUF <= 500_000)
    return _make_sc_kernel(N, D, C)(lb, ub, node_id.astype(jnp.int32))


# speedup vs baseline: 1.1258x; 1.0035x over previous
import functools

import jax
import jax.numpy as jnp
from jax import lax
from jax.experimental import pallas as pl
from jax.experimental.pallas import tpu as pltpu
from jax.experimental.pallas import tpu_sc as plsc

_NUM_CORES = 2
_NUM_SUBCORES = 16
_LANES = 16
_NBUF = 3


@functools.cache
def _make_sc_kernel(N: int, D: int, C: int):
    NW = _NUM_CORES * _NUM_SUBCORES
    NCHUNKS = N // C
    TMAX = (NCHUNKS + NW - 1) // NW
    NTB = (TMAX + 1 + _NBUF - 1) // _NBUF
    JV = D // _LANES

    mesh = plsc.VectorSubcoreMesh(
        core_axis_name="c", subcore_axis_name="s",
        num_cores=_NUM_CORES, num_subcores=_NUM_SUBCORES)

    def body(lb_hbm, ub_hbm, nid_hbm, out_hbm, idx_v, lb_v, ub_v,
             gsem, ssem, isem):
        wid = lax.axis_index("s") * _NUM_CORES + lax.axis_index("c")

        def in_range(t):
            return (t * NW + wid) < NCHUNKS

        def idx_desc(t, p):
            g = t * NW + wid
            return pltpu.make_async_copy(nid_hbm.at[pl.ds(g * C, C)],
                                         idx_v.at[p], isem.at[p])

        def start_idx(t, p):
            @pl.when(in_range(t))
            def _():
                idx_desc(t, p).start()

        def wait_idx(t, p):
            @pl.when(in_range(t))
            def _():
                idx_desc(t, p).wait()

        def gather_descs(p):
            return (pltpu.make_async_copy(lb_hbm.at[idx_v.at[p]], lb_v.at[p],
                                          gsem.at[p]),
                    pltpu.make_async_copy(ub_hbm.at[idx_v.at[p]], ub_v.at[p],
                                          gsem.at[p]))

        def scatter_descs(g, p):
            base = g * C
            return (pltpu.make_async_copy(lb_v.at[p],
                                          out_hbm.at[0, pl.ds(base, C)],
                                          ssem.at[p]),
                    pltpu.make_async_copy(ub_v.at[p],
                                          out_hbm.at[1, pl.ds(base, C)],
                                          ssem.at[p]))

        def start_gather(t, p, which=None):
            @pl.when(in_range(t))
            def _():
                descs = gather_descs(p)
                if which is None:
                    for d in descs:
                        d.start()
                else:
                    descs[which].start()

        def wait_gather(t, p):
            @pl.when(in_range(t))
            def _():
                for d in gather_descs(p):
                    d.wait()

        H0 = (C // 2 + 7) // 8 * 8
        SPLITS = tuple((lo, min(lo + 8, C)) for lo in range(0, C, 8)) \
            if C % 8 == 0 else ((0, H0), (H0, C))

        def start_scatter_half(t, p, h):
            @pl.when(in_range(t))
            def _():
                g = t * NW + wid
                lo, hi = SPLITS[h]
                for src, plane in ((lb_v, 0), (ub_v, 1)):
                    pltpu.make_async_copy(
                        src.at[p, pl.ds(lo, hi - lo)],
                        out_hbm.at[plane, pl.ds(g * C + lo, hi - lo)],
                        ssem.at[p]).start()

        def wait_scatter(t, p):
            @pl.when((t >= 0) & in_range(t))
            def _():
                g = t * NW + wid
                for d in scatter_descs(g, p):
                    d.wait()

        def compute_half(t, p, h):
            @pl.when(in_range(t))
            def _():
                def row_body(r, carry):
                    for j in range(JV):
                        sl = pl.ds(j * _LANES, _LANES)
                        l = lb_v[p, r, sl]
                        u = ub_v[p, r, sl]
                        zero = jnp.zeros_like(l)
                        ub_v[p, r, sl] = jnp.where(l == zero, zero,
                                                   jnp.maximum(u, zero))
                        lb_v[p, r, sl] = jnp.where((l + u) > zero, l, zero)
                    return carry

                lax.fori_loop(SPLITS[h][0], SPLITS[h][1], row_body, 0)

        start_idx(0, 0)
        start_idx(1, 1)
        start_idx(2, 2)
        wait_idx(0, 0)
        start_gather(0, 0)
        wait_idx(1, 1)
        start_gather(1, 1)

        def block(tb, carry):
            for p in range(_NBUF):
                t = tb * _NBUF + p
                wait_gather(t, p)
                pn = (p + 2) % _NBUF
                nseg = len(SPLITS)
                for h in range(nseg):
                    compute_half(t, p, h)
                    start_scatter_half(t, p, h)
                    if h == 0:
                        wait_scatter(t - 1, pn)
                        wait_idx(t + 2, pn)
                        start_gather(t + 2, pn, which=0)
                        start_idx(t + 3, p)
                    if h == nseg // 2:
                        start_gather(t + 2, pn, which=1)
            return carry

        lax.fori_loop(0, NTB, block, 0)

    return pl.kernel(
        body,
        out_type=jax.ShapeDtypeStruct((2, N, D), jnp.float32),
        mesh=mesh,
        scratch_types=[
            pltpu.VMEM((_NBUF, C), jnp.int32),
            pltpu.VMEM((_NBUF, C, D), jnp.float32),
            pltpu.VMEM((_NBUF, C, D), jnp.float32),
            pltpu.SemaphoreType.DMA((_NBUF,)),
            pltpu.SemaphoreType.DMA((_NBUF,)),
            pltpu.SemaphoreType.DMA((_NBUF,)),
        ],
    )


def kernel(lb, ub, node_id):
    N, D = lb.shape
    C = next(c for c in (40, 16, 8) if N % c == 0 and c * D * 4 * 2 * _NBUF <= 500_000)
    return _make_sc_kernel(N, D, C)(lb, ub, node_id.astype(jnp.int32))


# speedup vs baseline: 1.3744x; 1.2207x over previous
import functools

import jax
import jax.numpy as jnp
from jax import lax
from jax.experimental import pallas as pl
from jax.experimental.pallas import tpu as pltpu
from jax.experimental.pallas import tpu_sc as plsc

_NUM_CORES = 2
_NUM_SUBCORES = 16
_LANES = 16
_NBUF = 3


@functools.cache
def _make_sc_kernel(N: int, D: int, C: int):
    NW = _NUM_CORES * _NUM_SUBCORES
    NCHUNKS = N // C
    TMAX = (NCHUNKS + NW - 1) // NW
    NTB = (TMAX + 1 + _NBUF - 1) // _NBUF
    JV = D // _LANES

    mesh = plsc.VectorSubcoreMesh(
        core_axis_name="c", subcore_axis_name="s",
        num_cores=_NUM_CORES, num_subcores=_NUM_SUBCORES)

    def body(lb_hbm, ub_hbm, nid_hbm, out_hbm, idx_v, lb_v, ub_v,
             gsem, ssem, isem):
        wid = lax.axis_index("s") * _NUM_CORES + lax.axis_index("c")

        def in_range(t):
            return (t * NW + wid) < NCHUNKS

        def idx_desc(t, p):
            g = t * NW + wid
            return pltpu.make_async_copy(nid_hbm.at[pl.ds(g * C, C)],
                                         idx_v.at[p], isem.at[p])

        def start_idx(t, p):
            @pl.when(in_range(t))
            def _():
                idx_desc(t, p).start()

        def wait_idx(t, p):
            @pl.when(in_range(t))
            def _():
                idx_desc(t, p).wait()

        def gather_descs(p):
            return (pltpu.make_async_copy(lb_hbm.at[idx_v.at[p]], lb_v.at[p],
                                          gsem.at[p]),
                    pltpu.make_async_copy(ub_hbm.at[idx_v.at[p]], ub_v.at[p],
                                          gsem.at[p]))

        def scatter_descs(g, p):
            base = g * C
            return (pltpu.make_async_copy(lb_v.at[p],
                                          out_hbm.at[0, pl.ds(base, C)],
                                          ssem.at[p]),
                    pltpu.make_async_copy(ub_v.at[p],
                                          out_hbm.at[1, pl.ds(base, C)],
                                          ssem.at[p]))

        def start_gather(t, p, which=None):
            @pl.when(in_range(t))
            def _():
                descs = gather_descs(p)
                if which is None:
                    for d in descs:
                        d.start()
                else:
                    descs[which].start()

        def wait_gather(t, p):
            @pl.when(in_range(t))
            def _():
                for d in gather_descs(p):
                    d.wait()

        H0 = (C // 2 + 7) // 8 * 8
        SPLITS = tuple((lo, min(lo + 8, C)) for lo in range(0, C, 8)) \
            if C % 8 == 0 else ((0, H0), (H0, C))

        def start_scatter_half(t, p, h):
            @pl.when(in_range(t))
            def _():
                g = t * NW + wid
                lo, hi = SPLITS[h]
                for src, plane in ((lb_v, 0), (ub_v, 1)):
                    pltpu.make_async_copy(
                        src.at[p, pl.ds(lo, hi - lo)],
                        out_hbm.at[plane, pl.ds(g * C + lo, hi - lo)],
                        ssem.at[p]).start()

        def wait_scatter(t, p):
            @pl.when((t >= 0) & in_range(t))
            def _():
                g = t * NW + wid
                for d in scatter_descs(g, p):
                    d.wait()

        def compute_half(t, p, h):
            @pl.when(in_range(t))
            def _():
                def row_body(r, carry):
                    for j in range(JV):
                        sl = pl.ds(j * _LANES, _LANES)
                        l = lb_v[p, r, sl]
                        u = ub_v[p, r, sl]
                        zero = jnp.zeros_like(l)
                        ub_v[p, r, sl] = jnp.where(l == zero, zero,
                                                   jnp.maximum(u, zero))
                        lb_v[p, r, sl] = jnp.where((l + u) > zero, l, zero)
                    return carry

                lax.fori_loop(SPLITS[h][0], SPLITS[h][1], row_body, 0)

        start_idx(0, 0)
        start_idx(1, 1)
        start_idx(2, 2)
        wait_idx(0, 0)
        start_gather(0, 0)
        wait_idx(1, 1)
        start_gather(1, 1)

        def block(tb, carry):
            for p in range(_NBUF):
                t = tb * _NBUF + p
                wait_gather(t, p)
                pn = (p + 2) % _NBUF
                nseg = len(SPLITS)
                for h in range(nseg):
                    compute_half(t, p, h)
                    start_scatter_half(t, p, h)
                    if h == 0:
                        wait_scatter(t - 1, pn)
                        wait_idx(t + 2, pn)
                        start_gather(t + 2, pn, which=0)
                        start_idx(t + 3, p)
                    if h == nseg // 2:
                        start_gather(t + 2, pn, which=1)
            return carry

        lax.fori_loop(0, NTB, block, 0)

    return pl.kernel(
        body,
        out_type=jax.ShapeDtypeStruct((2, N, D), jnp.float32),
        mesh=mesh,
        scratch_types=[
            pltpu.VMEM((_NBUF, C), jnp.int32),
            pltpu.VMEM((_NBUF, C, D), jnp.float32),
            pltpu.VMEM((_NBUF, C, D), jnp.float32),
            pltpu.SemaphoreType.DMA((_NBUF,)),
            pltpu.SemaphoreType.DMA((_NBUF,)),
            pltpu.SemaphoreType.DMA((_NBUF,)),
        ],
    )


def kernel(lb, ub, node_id):
    N, D = lb.shape
    C = next(c for c in (16, 40, 8) if N % c == 0 and c * D * 4 * 2 * _NBUF <= 500_000)
    return _make_sc_kernel(N, D, C)(lb, ub, node_id.astype(jnp.int32))


# speedup vs baseline: 1.4217x; 1.0344x over previous
import functools

import jax
import jax.numpy as jnp
from jax import lax
from jax.experimental import pallas as pl
from jax.experimental.pallas import tpu as pltpu
from jax.experimental.pallas import tpu_sc as plsc

_NUM_CORES = 2
_NUM_SUBCORES = 16
_LANES = 16
_NBUF = 3


@functools.cache
def _make_sc_kernel(N: int, D: int, C: int):
    NW = _NUM_CORES * _NUM_SUBCORES
    NCHUNKS = N // C
    TMAX = (NCHUNKS + NW - 1) // NW
    NTB = (TMAX + 1 + _NBUF - 1) // _NBUF
    JV = D // _LANES

    mesh = plsc.VectorSubcoreMesh(
        core_axis_name="c", subcore_axis_name="s",
        num_cores=_NUM_CORES, num_subcores=_NUM_SUBCORES)

    def body(lb_hbm, ub_hbm, nid_hbm, out_hbm, idx_v, lb_v, ub_v,
             gsem, ssem, isem):
        wid = lax.axis_index("s") * _NUM_CORES + lax.axis_index("c")

        def in_range(t):
            return (t * NW + wid) < NCHUNKS

        def idx_desc(t, p):
            g = t * NW + wid
            return pltpu.make_async_copy(nid_hbm.at[pl.ds(g * C, C)],
                                         idx_v.at[p], isem.at[p])

        def start_idx(t, p):
            @pl.when(in_range(t))
            def _():
                idx_desc(t, p).start()

        def wait_idx(t, p):
            @pl.when(in_range(t))
            def _():
                idx_desc(t, p).wait()

        def gather_descs(p):
            return (pltpu.make_async_copy(lb_hbm.at[idx_v.at[p]], lb_v.at[p],
                                          gsem.at[p]),
                    pltpu.make_async_copy(ub_hbm.at[idx_v.at[p]], ub_v.at[p],
                                          gsem.at[p]))

        def scatter_descs(g, p):
            base = g * C
            return (pltpu.make_async_copy(lb_v.at[p],
                                          out_hbm.at[0, pl.ds(base, C)],
                                          ssem.at[p]),
                    pltpu.make_async_copy(ub_v.at[p],
                                          out_hbm.at[1, pl.ds(base, C)],
                                          ssem.at[p]))

        def start_gather(t, p, which=None):
            @pl.when(in_range(t))
            def _():
                descs = gather_descs(p)
                if which is None:
                    for d in descs:
                        d.start()
                else:
                    descs[which].start()

        def wait_gather(t, p):
            @pl.when(in_range(t))
            def _():
                for d in gather_descs(p):
                    d.wait()

        H0 = (C // 2 + 7) // 8 * 8
        SPLITS = tuple((lo, min(lo + 8, C)) for lo in range(0, C, 8)) \
            if C % 8 == 0 else ((0, H0), (H0, C))

        def start_scatter_half(t, p, h):
            @pl.when(in_range(t))
            def _():
                g = t * NW + wid
                lo, hi = SPLITS[h]
                for src, plane in ((lb_v, 0), (ub_v, 1)):
                    pltpu.make_async_copy(
                        src.at[p, pl.ds(lo, hi - lo)],
                        out_hbm.at[plane, pl.ds(g * C + lo, hi - lo)],
                        ssem.at[p]).start()

        def wait_scatter(t, p):
            @pl.when((t >= 0) & in_range(t))
            def _():
                g = t * NW + wid
                for d in scatter_descs(g, p):
                    d.wait()

        def compute_half(t, p, h):
            @pl.when(in_range(t))
            def _():
                def row_body(r, carry):
                    for j in range(JV):
                        sl = pl.ds(j * _LANES, _LANES)
                        l = lb_v[p, r, sl]
                        u = ub_v[p, r, sl]
                        zero = jnp.zeros_like(l)
                        ub_v[p, r, sl] = jnp.where(l == zero, zero,
                                                   jnp.maximum(u, zero))
                        lb_v[p, r, sl] = jnp.where((l + u) > zero, l, zero)
                    return carry

                lax.fori_loop(SPLITS[h][0], SPLITS[h][1], row_body, 0)

        start_idx(0, 0)
        start_idx(1, 1)
        start_idx(2, 2)
        wait_idx(0, 0)
        start_gather(0, 0)
        wait_idx(1, 1)
        start_gather(1, 1)

        def block(tb, carry):
            for p in range(_NBUF):
                t = tb * _NBUF + p
                wait_gather(t, p)
                pn = (p + 2) % _NBUF
                nseg = len(SPLITS)
                for h in range(nseg):
                    compute_half(t, p, h)
                    start_scatter_half(t, p, h)
                    if h == 0:
                        wait_scatter(t - 1, pn)
                        wait_idx(t + 2, pn)
                        start_gather(t + 2, pn, which=0)
                        start_idx(t + 3, p)
                    if h == nseg // 2:
                        start_gather(t + 2, pn, which=1)
            return carry

        lax.fori_loop(0, NTB, block, 0)

    return pl.kernel(
        body,
        out_type=jax.ShapeDtypeStruct((2, N, D), jnp.float32),
        mesh=mesh,
        scratch_types=[
            pltpu.VMEM((_NBUF, C), jnp.int32),
            pltpu.VMEM((_NBUF, C, D), jnp.float32),
            pltpu.VMEM((_NBUF, C, D), jnp.float32),
            pltpu.SemaphoreType.DMA((_NBUF,)),
            pltpu.SemaphoreType.DMA((_NBUF,)),
            pltpu.SemaphoreType.DMA((_NBUF,)),
        ],
    )


def kernel(lb, ub, node_id):
    N, D = lb.shape
    C = next(c for c in (8, 16, 40) if N % c == 0 and c * D * 4 * 2 * _NBUF <= 500_000)
    return _make_sc_kernel(N, D, C)(lb, ub, node_id.astype(jnp.int32))


# speedup vs baseline: 1.6674x; 1.1728x over previous
import functools

import jax
import jax.numpy as jnp
from jax import lax
from jax.experimental import pallas as pl
from jax.experimental.pallas import tpu as pltpu
from jax.experimental.pallas import tpu_sc as plsc

_NUM_CORES = 2
_NUM_SUBCORES = 16
_LANES = 16
_NBUF = 6


@functools.cache
def _make_sc_kernel(N: int, D: int, C: int):
    NW = _NUM_CORES * _NUM_SUBCORES
    NCHUNKS = N // C
    TMAX = (NCHUNKS + NW - 1) // NW
    NTB = (TMAX + 1 + _NBUF - 1) // _NBUF
    JV = D // _LANES

    mesh = plsc.VectorSubcoreMesh(
        core_axis_name="c", subcore_axis_name="s",
        num_cores=_NUM_CORES, num_subcores=_NUM_SUBCORES)

    def body(lb_hbm, ub_hbm, nid_hbm, out_hbm, idx_v, lb_v, ub_v,
             gsem, ssem, isem):
        wid = lax.axis_index("s") * _NUM_CORES + lax.axis_index("c")

        def in_range(t):
            return (t * NW + wid) < NCHUNKS

        def idx_desc(t, p):
            g = t * NW + wid
            return pltpu.make_async_copy(nid_hbm.at[pl.ds(g * C, C)],
                                         idx_v.at[p], isem.at[p])

        def start_idx(t, p):
            @pl.when(in_range(t))
            def _():
                idx_desc(t, p).start()

        def wait_idx(t, p):
            @pl.when(in_range(t))
            def _():
                idx_desc(t, p).wait()

        def gather_descs(p):
            return (pltpu.make_async_copy(lb_hbm.at[idx_v.at[p]], lb_v.at[p],
                                          gsem.at[p]),
                    pltpu.make_async_copy(ub_hbm.at[idx_v.at[p]], ub_v.at[p],
                                          gsem.at[p]))

        def scatter_descs(g, p):
            base = g * C
            return (pltpu.make_async_copy(lb_v.at[p],
                                          out_hbm.at[0, pl.ds(base, C)],
                                          ssem.at[p]),
                    pltpu.make_async_copy(ub_v.at[p],
                                          out_hbm.at[1, pl.ds(base, C)],
                                          ssem.at[p]))

        def start_gather(t, p, which=None):
            @pl.when(in_range(t))
            def _():
                descs = gather_descs(p)
                if which is None:
                    for d in descs:
                        d.start()
                else:
                    descs[which].start()

        def wait_gather(t, p):
            @pl.when(in_range(t))
            def _():
                for d in gather_descs(p):
                    d.wait()

        H0 = (C // 2 + 7) // 8 * 8
        SPLITS = tuple((lo, min(lo + 8, C)) for lo in range(0, C, 8)) \
            if C % 8 == 0 else ((0, H0), (H0, C))

        def start_scatter_half(t, p, h):
            @pl.when(in_range(t))
            def _():
                g = t * NW + wid
                lo, hi = SPLITS[h]
                for src, plane in ((lb_v, 0), (ub_v, 1)):
                    pltpu.make_async_copy(
                        src.at[p, pl.ds(lo, hi - lo)],
                        out_hbm.at[plane, pl.ds(g * C + lo, hi - lo)],
                        ssem.at[p]).start()

        def wait_scatter(t, p):
            @pl.when((t >= 0) & in_range(t))
            def _():
                g = t * NW + wid
                for d in scatter_descs(g, p):
                    d.wait()

        def compute_half(t, p, h):
            @pl.when(in_range(t))
            def _():
                def row_body(r, carry):
                    for j in range(JV):
                        sl = pl.ds(j * _LANES, _LANES)
                        l = lb_v[p, r, sl]
                        u = ub_v[p, r, sl]
                        zero = jnp.zeros_like(l)
                        ub_v[p, r, sl] = jnp.where(l == zero, zero,
                                                   jnp.maximum(u, zero))
                        lb_v[p, r, sl] = jnp.where((l + u) > zero, l, zero)
                    return carry

                lax.fori_loop(SPLITS[h][0], SPLITS[h][1], row_body, 0)

        L = _NBUF - 1

        for k in range(_NBUF):
            start_idx(k, k)
        for k in range(L):
            wait_idx(k, k)
            start_gather(k, k)

        def block(tb, carry):
            for p in range(_NBUF):
                t = tb * _NBUF + p
                wait_gather(t, p)
                pn = (p + L) % _NBUF
                nseg = len(SPLITS)
                for h in range(nseg):
                    compute_half(t, p, h)
                    start_scatter_half(t, p, h)
                    if h == 0:
                        wait_scatter(t - 1, pn)
                        wait_idx(t + L, pn)
                        start_gather(t + L, pn, which=0)
                        start_idx(t + L + 1, p)
                    if h == nseg // 2:
                        start_gather(t + L, pn, which=1)
            return carry

        lax.fori_loop(0, NTB, block, 0)

    return pl.kernel(
        body,
        out_type=jax.ShapeDtypeStruct((2, N, D), jnp.float32),
        mesh=mesh,
        scratch_types=[
            pltpu.VMEM((_NBUF, C), jnp.int32),
            pltpu.VMEM((_NBUF, C, D), jnp.float32),
            pltpu.VMEM((_NBUF, C, D), jnp.float32),
            pltpu.SemaphoreType.DMA((_NBUF,)),
            pltpu.SemaphoreType.DMA((_NBUF,)),
            pltpu.SemaphoreType.DMA((_NBUF,)),
        ],
    )


def kernel(lb, ub, node_id):
    N, D = lb.shape
    C = next(c for c in (8, 16, 40) if N % c == 0 and c * D * 4 * 2 * _NBUF <= 500_000)
    return _make_sc_kernel(N, D, C)(lb, ub, node_id.astype(jnp.int32))
